# baseline (speedup 1.0000x reference)
import functools
import math

import jax
import jax.numpy as jnp
from jax import lax
from jax.experimental import pallas as pl
from jax.experimental.pallas import tpu as pltpu
from jax.experimental.pallas import tpu_sc as plsc

VOCAB = 1000000
EMB = 64
BATCH = 16384
SEQ = 50
SCALE = math.sqrt(EMB)

NC = 2
NS = 16
NW = NC * NS
BCOLS = BATCH // NW
CHUNK = 256
NCHUNK = SEQ * 2
NBLK = (CHUNK // 16) * (EMB // 16)
SCR_STRIDE = 16 * 17

_mesh = plsc.VectorSubcoreMesh(core_axis_name="c", subcore_axis_name="s")


@functools.partial(
    pl.kernel,
    mesh=_mesh,
    out_type=jax.ShapeDtypeStruct((SEQ * 8, 128, 1024), jnp.float32),
    scratch_types=[
        pltpu.VMEM((CHUNK,), jnp.int32),
        pltpu.VMEM((CHUNK,), jnp.int32),
        pltpu.VMEM((CHUNK, EMB), jnp.float32),
        pltpu.VMEM((CHUNK, EMB), jnp.float32),
        pltpu.VMEM((2, 8, 1, 1024), jnp.float32),
        pltpu.VMEM((2, 8, 1, 1024), jnp.float32),
        pltpu.VMEM((NBLK * SCR_STRIDE,), jnp.float32),
        pltpu.SemaphoreType.DMA,
        pltpu.SemaphoreType.DMA,
        pltpu.SemaphoreType.DMA,
        pltpu.SemaphoreType.DMA,
        pltpu.SemaphoreType.DMA,
        pltpu.SemaphoreType.DMA,
    ],
    compiler_params=pltpu.CompilerParams(
        use_tc_tiling_on_sc=False, needs_layout_passes=False),
)
def _emb_lookup(tokens2_hbm, table_hbm, out_hbm, i0, i1, g0, g1, o0, o1,
                scr, si0, si1, sg0, sg1, ow0, ow1):
    wid = lax.axis_index("s") * NC + lax.axis_index("c")
    col0 = wid * BCOLS
    tc0 = wid * (BCOLS // 128)
    lane = lax.iota(jnp.int32, 16)
    i17 = lane * 17
    i17jj = [i17 + jj for jj in range(16)]

    bufs = ((i0, g0, o0, si0, sg0, ow0), (i1, g1, o1, si1, sg1, ow1))

    def start_idx(c, ibuf, sem):
        pltpu.async_copy(
            tokens2_hbm.at[c // 2, pl.ds(col0 + (c % 2) * CHUNK, CHUNK)],
            ibuf, sem)

    def wait_idx(ibuf, sem):
        pltpu.make_async_copy(
            tokens2_hbm.at[0, pl.ds(0, CHUNK)], ibuf, sem).wait()

    def start_gather(ibuf, gbuf, sem):
        pltpu.async_copy(table_hbm.at[ibuf], gbuf, sem)

    def wait_gather(gbuf, sem):
        pltpu.make_async_copy(table_hbm.at[i0], gbuf, sem).wait()

    def start_write(c, obuf, sem):
        s = c // 2
        tc = tc0 + (c % 2) * 2
        for k in range(2):
            pltpu.async_copy(
                obuf.at[k],
                out_hbm.at[pl.ds(s * 8, 8), pl.ds(tc + k, 1), :], sem)

    def wait_write(obuf, sem):
        for k in range(2):
            pltpu.make_async_copy(
                obuf.at[k], out_hbm.at[pl.ds(0, 8), pl.ds(0, 1), :],
                sem).wait()

    start_idx(0, i0, si0)
    start_idx(1, i1, si1)
    wait_idx(i0, si0)
    start_gather(i0, g0, sg0)

    def loop_body(it, carry):
        for b in range(2):
            c = it * 2 + b
            ibuf, gbuf, obuf, sis, sgs, ows = bufs[b]
            nibuf, ngbuf, _, nsis, nsgs, _ = bufs[1 - b]
            wait_gather(gbuf, sgs)

            @pl.when(c + 2 < NCHUNK)
            def _():
                start_idx(c + 2, ibuf, sis)

            @pl.when(c >= 2)
            def _():
                wait_write(obuf, ows)

            @plsc.parallel_loop(0, NBLK, 1, unroll=2)
            def _stage(blk):
                bb = lax.bitwise_and(blk, 15)
                eg = lax.shift_right_logical(blk, 4)
                e0 = eg * 16
                sbase = blk * SCR_STRIDE
                for j in range(16):
                    v = gbuf[bb * 16 + j, pl.ds(e0, 16)]
                    scr[pl.ds(sbase + j * 17, 16)] = v

            @plsc.parallel_loop(0, NBLK, 1, unroll=2)
            def _drain(blk):
                bb = lax.bitwise_and(blk, 15)
                eg = lax.shift_right_logical(blk, 4)
                sbase = blk * SCR_STRIDE
                k = lax.shift_right_logical(bb, 3)
                c0 = lax.bitwise_and(bb, 7) * 16
                sub = scr.at[pl.ds(sbase, SCR_STRIDE)]
                for jj in range(16):
                    vt = plsc.load_gather(sub, [i17jj[jj]])
                    tr2 = 2 * eg + jj // 8
                    off = (jj % 8) * 128 + c0
                    obuf[k, tr2, 0, pl.ds(off, 16)] = vt * SCALE

            start_write(c, obuf, ows)

            @pl.when(c + 1 < NCHUNK)
            def _():
                wait_idx(nibuf, nsis)
                start_gather(nibuf, ngbuf, nsgs)

        return carry

    lax.fori_loop(0, NCHUNK // 2, loop_body, 0)
    wait_write(o0, ow0)
    wait_write(o1, ow1)


def kernel(tokens, table):
    out5 = _emb_lookup(tokens.T, table)
    out5 = out5.reshape(SEQ, 8, 128, 8, 128)
    return out5.transpose(2, 4, 0, 1, 3).reshape(BATCH, SEQ, EMB)


# speedup vs baseline: 1.0018x; 1.0018x over previous
import functools
import math

import jax
import jax.numpy as jnp
from jax import lax
from jax.experimental import pallas as pl
from jax.experimental.pallas import tpu as pltpu
from jax.experimental.pallas import tpu_sc as plsc

VOCAB = 1000000
EMB = 64
BATCH = 16384
SEQ = 50
SCALE = math.sqrt(EMB)

NC = 2
NS = 16
NW = NC * NS
BCOLS = BATCH // NW
CHUNK = 256
NCHUNK = SEQ * 2
NBLK = (CHUNK // 16) * (EMB // 16)
SCR_STRIDE = 16 * 17

_mesh = plsc.VectorSubcoreMesh(core_axis_name="c", subcore_axis_name="s")


@functools.partial(
    pl.kernel,
    mesh=_mesh,
    out_type=jax.ShapeDtypeStruct((SEQ * 8, 128, 1024), jnp.float32),
    scratch_types=[
        pltpu.VMEM((CHUNK,), jnp.int32),
        pltpu.VMEM((CHUNK,), jnp.int32),
        pltpu.VMEM((CHUNK, EMB), jnp.float32),
        pltpu.VMEM((CHUNK, EMB), jnp.float32),
        pltpu.VMEM((2, 8, 1, 1024), jnp.float32),
        pltpu.VMEM((2, 8, 1, 1024), jnp.float32),
        pltpu.VMEM((NBLK * SCR_STRIDE,), jnp.float32),
        pltpu.SemaphoreType.DMA,
        pltpu.SemaphoreType.DMA,
        pltpu.SemaphoreType.DMA,
        pltpu.SemaphoreType.DMA,
        pltpu.SemaphoreType.DMA,
        pltpu.SemaphoreType.DMA,
    ],
    compiler_params=pltpu.CompilerParams(
        use_tc_tiling_on_sc=False, needs_layout_passes=False),
)
def _emb_lookup(tokens2_hbm, table_hbm, out_hbm, i0, i1, g0, g1, o0, o1,
                scr, si0, si1, sg0, sg1, ow0, ow1):
    wid = lax.axis_index("s") * NC + lax.axis_index("c")
    col0 = wid * BCOLS
    tc0 = wid * (BCOLS // 128)
    lane = lax.iota(jnp.int32, 16)
    i17 = lane * 17
    i17jj = [i17 + jj for jj in range(16)]

    bufs = ((i0, g0, o0, si0, sg0, ow0), (i1, g1, o1, si1, sg1, ow1))

    def start_idx(c, ibuf, sem):
        pltpu.async_copy(
            tokens2_hbm.at[c // 2, pl.ds(col0 + (c % 2) * CHUNK, CHUNK)],
            ibuf, sem)

    def wait_idx(ibuf, sem):
        pltpu.make_async_copy(
            tokens2_hbm.at[0, pl.ds(0, CHUNK)], ibuf, sem).wait()

    def start_gather(ibuf, gbuf, sem):
        pltpu.async_copy(table_hbm.at[ibuf], gbuf, sem)

    def wait_gather(gbuf, sem):
        pltpu.make_async_copy(table_hbm.at[i0], gbuf, sem).wait()

    def start_write(c, obuf, sem):
        s = c // 2
        tc = tc0 + (c % 2) * 2
        for k in range(2):
            pltpu.async_copy(
                obuf.at[k],
                out_hbm.at[pl.ds(s * 8, 8), pl.ds(tc + k, 1), :], sem)

    def wait_write(obuf, sem):
        for k in range(2):
            pltpu.make_async_copy(
                obuf.at[k], out_hbm.at[pl.ds(0, 8), pl.ds(0, 1), :],
                sem).wait()

    start_idx(0, i0, si0)
    start_idx(1, i1, si1)
    wait_idx(i0, si0)
    start_gather(i0, g0, sg0)

    def loop_body(it, carry):
        for b in range(2):
            c = it * 2 + b
            ibuf, gbuf, obuf, sis, sgs, ows = bufs[b]
            nibuf, ngbuf, _, nsis, nsgs, _ = bufs[1 - b]
            wait_gather(gbuf, sgs)

            @pl.when(c + 2 < NCHUNK)
            def _():
                start_idx(c + 2, ibuf, sis)

            @plsc.parallel_loop(0, NBLK, 1, unroll=2)
            def _stage(blk):
                bb = lax.bitwise_and(blk, 15)
                eg = lax.shift_right_logical(blk, 4)
                e0 = eg * 16
                sbase = blk * SCR_STRIDE
                for j in range(16):
                    v = gbuf[bb * 16 + j, pl.ds(e0, 16)]
                    scr[pl.ds(sbase + j * 17, 16)] = v

            @pl.when(c >= 2)
            def _():
                wait_write(obuf, ows)

            @plsc.parallel_loop(0, NBLK, 1, unroll=2)
            def _drain(blk):
                bb = lax.bitwise_and(blk, 15)
                eg = lax.shift_right_logical(blk, 4)
                sbase = blk * SCR_STRIDE
                k = lax.shift_right_logical(bb, 3)
                c0 = lax.bitwise_and(bb, 7) * 16
                sub = scr.at[pl.ds(sbase, SCR_STRIDE)]
                for jj in range(16):
                    vt = plsc.load_gather(sub, [i17jj[jj]])
                    tr2 = 2 * eg + jj // 8
                    off = (jj % 8) * 128 + c0
                    obuf[k, tr2, 0, pl.ds(off, 16)] = vt * SCALE

            start_write(c, obuf, ows)

            @pl.when(c + 1 < NCHUNK)
            def _():
                wait_idx(nibuf, nsis)
                start_gather(nibuf, ngbuf, nsgs)

        return carry

    lax.fori_loop(0, NCHUNK // 2, loop_body, 0)
    wait_write(o0, ow0)
    wait_write(o1, ow1)


def kernel(tokens, table):
    out5 = _emb_lookup(tokens.T, table)
    out5 = out5.reshape(SEQ, 8, 128, 8, 128)
    return out5.transpose(2, 4, 0, 1, 3).reshape(BATCH, SEQ, EMB)


# speedup vs baseline: 1.0764x; 1.0745x over previous
import functools
import math

import jax
import jax.numpy as jnp
from jax import lax
from jax.experimental import pallas as pl
from jax.experimental.pallas import tpu as pltpu
from jax.experimental.pallas import tpu_sc as plsc

VOCAB = 1000000
EMB = 64
BATCH = 16384
SEQ = 50
SCALE = math.sqrt(EMB)

NC = 2
NS = 16
NW = NC * NS
BCOLS = BATCH // NW
CHUNK = 256
NCHUNK = SEQ * 2
NBLK = (CHUNK // 16) * (EMB // 16)
SCR_STRIDE = 16 * 17

_mesh = plsc.VectorSubcoreMesh(core_axis_name="c", subcore_axis_name="s")


@functools.partial(
    pl.kernel,
    mesh=_mesh,
    out_type=jax.ShapeDtypeStruct((SEQ * 8, 128, 1024), jnp.float32),
    scratch_types=[
        pltpu.VMEM((CHUNK,), jnp.int32),
        pltpu.VMEM((CHUNK,), jnp.int32),
        pltpu.VMEM((CHUNK, EMB), jnp.float32),
        pltpu.VMEM((CHUNK, EMB), jnp.float32),
        pltpu.VMEM((2, 8, 1, 1024), jnp.float32),
        pltpu.VMEM((2, 8, 1, 1024), jnp.float32),
        pltpu.VMEM((NBLK * SCR_STRIDE,), jnp.float32),
        pltpu.SemaphoreType.DMA,
        pltpu.SemaphoreType.DMA,
        pltpu.SemaphoreType.DMA,
        pltpu.SemaphoreType.DMA,
        pltpu.SemaphoreType.DMA,
        pltpu.SemaphoreType.DMA,
    ],
    compiler_params=pltpu.CompilerParams(
        use_tc_tiling_on_sc=False, needs_layout_passes=False),
)
def _emb_lookup(tokens2_hbm, table_hbm, out_hbm, i0, i1, g0, g1, o0, o1,
                scr, si0, si1, sg0, sg1, ow0, ow1):
    wid = lax.axis_index("s") * NC + lax.axis_index("c")
    col0 = wid * BCOLS
    tc0 = wid * (BCOLS // 128)
    lane = lax.iota(jnp.int32, 16)
    i17 = lane * 17
    i17jj = [i17 + jj for jj in range(16)]

    bufs = ((i0, g0, o0, si0, sg0, ow0), (i1, g1, o1, si1, sg1, ow1))

    def start_idx(c, ibuf, sem):
        pltpu.async_copy(
            tokens2_hbm.at[c // 2, pl.ds(col0 + (c % 2) * CHUNK, CHUNK)],
            ibuf, sem)

    def wait_idx(ibuf, sem):
        pltpu.make_async_copy(
            tokens2_hbm.at[0, pl.ds(0, CHUNK)], ibuf, sem).wait()

    def start_gather(ibuf, gbuf, sem):
        pltpu.async_copy(table_hbm.at[ibuf], gbuf, sem)

    def wait_gather(gbuf, sem):
        pltpu.make_async_copy(table_hbm.at[i0], gbuf, sem).wait()

    def start_write(c, obuf, sem):
        s = c // 2
        tc = tc0 + (c % 2) * 2
        for k in range(2):
            pltpu.async_copy(
                obuf.at[k],
                out_hbm.at[pl.ds(s * 8, 8), pl.ds(tc + k, 1), :], sem)

    def wait_write(obuf, sem):
        for k in range(2):
            pltpu.make_async_copy(
                obuf.at[k], out_hbm.at[pl.ds(0, 8), pl.ds(0, 1), :],
                sem).wait()

    start_idx(0, i0, si0)
    start_idx(1, i1, si1)
    wait_idx(i0, si0)
    start_gather(i0, g0, sg0)

    def loop_body(it, carry):
        for b in range(2):
            c = it * 2 + b
            ibuf, gbuf, obuf, sis, sgs, ows = bufs[b]
            nibuf, ngbuf, _, nsis, nsgs, _ = bufs[1 - b]
            wait_gather(gbuf, sgs)

            @pl.when(c + 2 < NCHUNK)
            def _():
                start_idx(c + 2, ibuf, sis)

            @pl.when(c >= 2)
            def _():
                wait_write(obuf, ows)

            @plsc.parallel_loop(0, NBLK, 1, unroll=4)
            def _transpose(blk):
                bb = lax.bitwise_and(blk, 15)
                eg = lax.shift_right_logical(blk, 4)
                e0 = eg * 16
                sbase = blk * SCR_STRIDE
                k = lax.shift_right_logical(bb, 3)
                c0 = lax.bitwise_and(bb, 7) * 16
                for j in range(16):
                    v = gbuf[bb * 16 + j, pl.ds(e0, 16)]
                    scr[pl.ds(sbase + j * 17, 16)] = v
                sub = scr.at[pl.ds(sbase, SCR_STRIDE)]
                for jj in range(16):
                    vt = plsc.load_gather(sub, [i17jj[jj]])
                    tr2 = 2 * eg + jj // 8
                    off = (jj % 8) * 128 + c0
                    obuf[k, tr2, 0, pl.ds(off, 16)] = vt * SCALE

            start_write(c, obuf, ows)

            @pl.when(c + 1 < NCHUNK)
            def _():
                wait_idx(nibuf, nsis)
                start_gather(nibuf, ngbuf, nsgs)

        return carry

    lax.fori_loop(0, NCHUNK // 2, loop_body, 0)
    wait_write(o0, ow0)
    wait_write(o1, ow1)


def kernel(tokens, table):
    out5 = _emb_lookup(tokens.T, table)
    out5 = out5.reshape(SEQ, 8, 128, 8, 128)
    return out5.transpose(2, 4, 0, 1, 3).reshape(BATCH, SEQ, EMB)


# speedup vs baseline: 1.1051x; 1.0267x over previous
import functools
import math

import jax
import jax.numpy as jnp
from jax import lax
from jax.experimental import pallas as pl
from jax.experimental.pallas import tpu as pltpu
from jax.experimental.pallas import tpu_sc as plsc

VOCAB = 1000000
EMB = 64
BATCH = 16384
SEQ = 50
SCALE = math.sqrt(EMB)

NC = 2
NS = 16
NW = NC * NS
BCOLS = BATCH // NW
CHUNK = 256
NCHUNK = SEQ * 2
NBLK = (CHUNK // 16) * (EMB // 16)
SCR_STRIDE = 16 * 17

_mesh = plsc.VectorSubcoreMesh(core_axis_name="c", subcore_axis_name="s")


@functools.partial(
    pl.kernel,
    mesh=_mesh,
    out_type=jax.ShapeDtypeStruct((SEQ * 8, 128, 1024), jnp.float32),
    scratch_types=[
        pltpu.VMEM((CHUNK,), jnp.int32),
        pltpu.VMEM((CHUNK,), jnp.int32),
        pltpu.VMEM((CHUNK, EMB), jnp.float32),
        pltpu.VMEM((CHUNK, EMB), jnp.float32),
        pltpu.VMEM((2, 8, 1, 1024), jnp.float32),
        pltpu.VMEM((2, 8, 1, 1024), jnp.float32),
        pltpu.VMEM((NBLK * SCR_STRIDE,), jnp.float32),
        pltpu.SemaphoreType.DMA,
        pltpu.SemaphoreType.DMA,
        pltpu.SemaphoreType.DMA,
        pltpu.SemaphoreType.DMA,
        pltpu.SemaphoreType.DMA,
        pltpu.SemaphoreType.DMA,
    ],
    compiler_params=pltpu.CompilerParams(
        use_tc_tiling_on_sc=False, needs_layout_passes=False),
)
def _emb_lookup(tokens2_hbm, table_hbm, out_hbm, i0, i1, g0, g1, o0, o1,
                scr, si0, si1, sg0, sg1, ow0, ow1):
    wid = lax.axis_index("s") * NC + lax.axis_index("c")
    col0 = wid * BCOLS
    tc0 = wid * (BCOLS // 128)
    lane = lax.iota(jnp.int32, 16)
    i17 = lane * 17
    i17jj = [i17 + jj for jj in range(16)]

    bufs = ((i0, g0, o0, si0, sg0, ow0), (i1, g1, o1, si1, sg1, ow1))

    def start_idx(c, ibuf, sem):
        pltpu.async_copy(
            tokens2_hbm.at[c // 2, pl.ds(col0 + (c % 2) * CHUNK, CHUNK)],
            ibuf, sem)

    def wait_idx(ibuf, sem):
        pltpu.make_async_copy(
            tokens2_hbm.at[0, pl.ds(0, CHUNK)], ibuf, sem).wait()

    def start_gather(ibuf, gbuf, sem):
        pltpu.async_copy(table_hbm.at[ibuf], gbuf, sem)

    def wait_gather(gbuf, sem):
        pltpu.make_async_copy(table_hbm.at[i0], gbuf, sem).wait()

    def start_write(c, obuf, sem):
        s = c // 2
        tc = tc0 + (c % 2) * 2
        for k in range(2):
            pltpu.async_copy(
                obuf.at[k],
                out_hbm.at[pl.ds(s * 8, 8), pl.ds(tc + k, 1), :], sem)

    def wait_write(obuf, sem):
        for k in range(2):
            pltpu.make_async_copy(
                obuf.at[k], out_hbm.at[pl.ds(0, 8), pl.ds(0, 1), :],
                sem).wait()

    start_idx(0, i0, si0)
    start_idx(1, i1, si1)
    wait_idx(i0, si0)
    start_gather(i0, g0, sg0)

    def loop_body(it, carry):
        for b in range(2):
            c = it * 2 + b
            ibuf, gbuf, obuf, sis, sgs, ows = bufs[b]
            nibuf, ngbuf, _, nsis, nsgs, _ = bufs[1 - b]
            wait_gather(gbuf, sgs)

            @pl.when(c + 2 < NCHUNK)
            def _():
                start_idx(c + 2, ibuf, sis)

            @pl.when(c >= 2)
            def _():
                wait_write(obuf, ows)

            @plsc.parallel_loop(0, NBLK, 1, unroll=8)
            def _transpose(blk):
                bb = lax.bitwise_and(blk, 15)
                eg = lax.shift_right_logical(blk, 4)
                e0 = eg * 16
                sbase = blk * SCR_STRIDE
                k = lax.shift_right_logical(bb, 3)
                c0 = lax.bitwise_and(bb, 7) * 16
                for j in range(16):
                    v = gbuf[bb * 16 + j, pl.ds(e0, 16)]
                    scr[pl.ds(sbase + j * 17, 16)] = v
                sub = scr.at[pl.ds(sbase, SCR_STRIDE)]
                for jj in range(16):
                    vt = plsc.load_gather(sub, [i17jj[jj]])
                    tr2 = 2 * eg + jj // 8
                    off = (jj % 8) * 128 + c0
                    obuf[k, tr2, 0, pl.ds(off, 16)] = vt * SCALE

            start_write(c, obuf, ows)

            @pl.when(c + 1 < NCHUNK)
            def _():
                wait_idx(nibuf, nsis)
                start_gather(nibuf, ngbuf, nsgs)

        return carry

    lax.fori_loop(0, NCHUNK // 2, loop_body, 0)
    wait_write(o0, ow0)
    wait_write(o1, ow1)


def kernel(tokens, table):
    out5 = _emb_lookup(tokens.T, table)
    out5 = out5.reshape(SEQ, 8, 128, 8, 128)
    return out5.transpose(2, 4, 0, 1, 3).reshape(BATCH, SEQ, EMB)


# speedup vs baseline: 1.1207x; 1.0141x over previous
import functools
import math

import jax
import jax.numpy as jnp
from jax import lax
from jax.experimental import pallas as pl
from jax.experimental.pallas import tpu as pltpu
from jax.experimental.pallas import tpu_sc as plsc

VOCAB = 1000000
EMB = 64
BATCH = 16384
SEQ = 50
SCALE = math.sqrt(EMB)

NC = 2
NS = 16
NW = NC * NS
BCOLS = BATCH // NW
CHUNK = 256
NCHUNK = SEQ * 2
NBLK = (CHUNK // 16) * (EMB // 16)
SCR_STRIDE = 16 * 17

_mesh = plsc.VectorSubcoreMesh(core_axis_name="c", subcore_axis_name="s")


@functools.partial(
    pl.kernel,
    mesh=_mesh,
    out_type=jax.ShapeDtypeStruct((SEQ * 8, 128, 1024), jnp.float32),
    scratch_types=[
        pltpu.VMEM((CHUNK,), jnp.int32),
        pltpu.VMEM((CHUNK,), jnp.int32),
        pltpu.VMEM((CHUNK, EMB), jnp.float32),
        pltpu.VMEM((CHUNK, EMB), jnp.float32),
        pltpu.VMEM((2, 8, 1, 1024), jnp.float32),
        pltpu.VMEM((2, 8, 1, 1024), jnp.float32),
        pltpu.VMEM((NBLK * SCR_STRIDE,), jnp.float32),
        pltpu.SemaphoreType.DMA,
        pltpu.SemaphoreType.DMA,
        pltpu.SemaphoreType.DMA,
        pltpu.SemaphoreType.DMA,
        pltpu.SemaphoreType.DMA,
        pltpu.SemaphoreType.DMA,
    ],
    compiler_params=pltpu.CompilerParams(
        use_tc_tiling_on_sc=False, needs_layout_passes=False),
)
def _emb_lookup(tokens2_hbm, table_hbm, out_hbm, i0, i1, g0, g1, o0, o1,
                scr, si0, si1, sg0, sg1, ow0, ow1):
    wid = lax.axis_index("s") * NC + lax.axis_index("c")
    col0 = wid * BCOLS
    tc0 = wid * (BCOLS // 128)
    lane = lax.iota(jnp.int32, 16)
    i17 = lane * 17
    i17jj = [i17 + jj for jj in range(16)]

    bufs = ((i0, g0, o0, si0, sg0, ow0), (i1, g1, o1, si1, sg1, ow1))

    def start_idx(c, ibuf, sem):
        pltpu.async_copy(
            tokens2_hbm.at[c // 2, pl.ds(col0 + (c % 2) * CHUNK, CHUNK)],
            ibuf, sem)

    def wait_idx(ibuf, sem):
        pltpu.make_async_copy(
            tokens2_hbm.at[0, pl.ds(0, CHUNK)], ibuf, sem).wait()

    def start_gather(ibuf, gbuf, sem):
        pltpu.async_copy(table_hbm.at[ibuf], gbuf, sem)

    def wait_gather(gbuf, sem):
        pltpu.make_async_copy(table_hbm.at[i0], gbuf, sem).wait()

    def start_write(c, obuf, sem):
        s = c // 2
        tc = tc0 + (c % 2) * 2
        for k in range(2):
            pltpu.async_copy(
                obuf.at[k],
                out_hbm.at[pl.ds(s * 8, 8), pl.ds(tc + k, 1), :], sem)

    def wait_write(obuf, sem):
        for k in range(2):
            pltpu.make_async_copy(
                obuf.at[k], out_hbm.at[pl.ds(0, 8), pl.ds(0, 1), :],
                sem).wait()

    start_idx(0, i0, si0)
    start_idx(1, i1, si1)
    wait_idx(i0, si0)
    start_gather(i0, g0, sg0)

    def loop_body(it, carry):
        for b in range(2):
            c = it * 2 + b
            ibuf, gbuf, obuf, sis, sgs, ows = bufs[b]
            nibuf, ngbuf, _, nsis, nsgs, _ = bufs[1 - b]
            wait_gather(gbuf, sgs)

            @pl.when(c + 2 < NCHUNK)
            def _():
                start_idx(c + 2, ibuf, sis)

            @pl.when(c >= 2)
            def _():
                wait_write(obuf, ows)

            @plsc.parallel_loop(0, NBLK, 1, unroll=16)
            def _transpose(blk):
                bb = lax.bitwise_and(blk, 15)
                eg = lax.shift_right_logical(blk, 4)
                e0 = eg * 16
                sbase = blk * SCR_STRIDE
                k = lax.shift_right_logical(bb, 3)
                c0 = lax.bitwise_and(bb, 7) * 16
                for j in range(16):
                    v = gbuf[bb * 16 + j, pl.ds(e0, 16)]
                    scr[pl.ds(sbase + j * 17, 16)] = v
                sub = scr.at[pl.ds(sbase, SCR_STRIDE)]
                for jj in range(16):
                    vt = plsc.load_gather(sub, [i17jj[jj]])
                    tr2 = 2 * eg + jj // 8
                    off = (jj % 8) * 128 + c0
                    obuf[k, tr2, 0, pl.ds(off, 16)] = vt * SCALE

            start_write(c, obuf, ows)

            @pl.when(c + 1 < NCHUNK)
            def _():
                wait_idx(nibuf, nsis)
                start_gather(nibuf, ngbuf, nsgs)

        return carry

    lax.fori_loop(0, NCHUNK // 2, loop_body, 0)
    wait_write(o0, ow0)
    wait_write(o1, ow1)


def kernel(tokens, table):
    out5 = _emb_lookup(tokens.T, table)
    out5 = out5.reshape(SEQ, 8, 128, 8, 128)
    return out5.transpose(2, 4, 0, 1, 3).reshape(BATCH, SEQ, EMB)
